# baseline (speedup 1.0000x reference)
import functools

import jax
import jax.numpy as jnp
from jax import lax
from jax.experimental import pallas as pl
from jax.experimental.pallas import tpu as pltpu
from jax.experimental.pallas import tpu_sc as plsc

_NC = 2
_NS = 16
_NW = _NC * _NS
_LANES = 16
_CHUNK = 128


def _scores_kernel(B, D, users_hbm, items_hbm, ut_hbm, it_hbm, out_hbm,
                   uidx, iidx, urows, irows, oscr, usem, isem):
    bpw = B // _NW
    nchunk = bpw // _CHUNK
    wid = lax.axis_index("s") * _NC + lax.axis_index("c")
    base = wid * bpw

    for j in range(nchunk):
        pltpu.sync_copy(users_hbm.at[pl.ds(base + j * _CHUNK, _CHUNK)],
                        uidx.at[j])
        pltpu.sync_copy(items_hbm.at[pl.ds(base + j * _CHUNK, _CHUNK)],
                        iidx.at[j])

    copies = []
    for j in range(nchunk):
        copies.append(pltpu.async_copy(ut_hbm.at[uidx.at[j]], urows.at[j],
                                       usem))
        copies.append(pltpu.async_copy(it_hbm.at[iidx.at[j]], irows.at[j],
                                       isem))
    for c in copies:
        c.wait()

    lanes = lax.iota(jnp.int32, _LANES)
    groups_per_chunk = _CHUNK // _LANES

    def group_body(g, _):
        chunk = g // groups_per_chunk
        within = (g % groups_per_chunk) * _LANES + lanes
        chunk_v = jnp.full((_LANES,), chunk, jnp.int32)
        acc = jnp.zeros((_LANES,), jnp.float32)
        for d in range(D):
            col = jnp.full((_LANES,), d, jnp.int32)
            uv = plsc.load_gather(urows, [chunk_v, within, col])
            iv = plsc.load_gather(irows, [chunk_v, within, col])
            acc = acc + uv * iv
        sig = 1.0 / (1.0 + jnp.exp(-acc))
        oscr[pl.ds(g * _LANES, _LANES)] = sig
        return 0

    lax.fori_loop(0, bpw // _LANES, group_body, 0)
    pltpu.sync_copy(oscr, out_hbm.at[pl.ds(base, bpw)])


def kernel(users, items, user_table, item_table):
    B = users.shape[0]
    D = user_table.shape[1]
    bpw = B // _NW
    nchunk = bpw // _CHUNK
    mesh = plsc.VectorSubcoreMesh(core_axis_name="c", subcore_axis_name="s")

    run = functools.partial(
        pl.kernel,
        mesh=mesh,
        compiler_params=pltpu.CompilerParams(
            needs_layout_passes=False, use_tc_tiling_on_sc=False),
        out_type=jax.ShapeDtypeStruct((B,), jnp.float32),
        scratch_types=[
            pltpu.VMEM((nchunk, _CHUNK), jnp.int32),
            pltpu.VMEM((nchunk, _CHUNK), jnp.int32),
            pltpu.VMEM((nchunk, _CHUNK, D), jnp.float32),
            pltpu.VMEM((nchunk, _CHUNK, D), jnp.float32),
            pltpu.VMEM((bpw,), jnp.float32),
            pltpu.SemaphoreType.DMA,
            pltpu.SemaphoreType.DMA,
        ],
    )(functools.partial(_scores_kernel, B, D))
    return run(users, items, user_table, item_table)
